# Initial kernel scaffold; baseline (speedup 1.0000x reference)
import jax
import jax.numpy as jnp
from jax.experimental import pallas as pl


def kernel(expr, jaw_quat_weight, xyz, feat_lines_x, feat_lines_y, feat_lines_z, v0, g0, b0, v1, g1, b1, v2, g2, b2):
    raise NotImplementedError("write your pallas kernel here")


# speedup vs baseline: 77.7079x; 77.7079x over previous
import jax
import jax.numpy as jnp
from jax.experimental import pallas as pl
from jax.experimental.pallas import tpu as pltpu

_EXPR = 80
_L = 64
_C = 32
_B = 2048


def _body(E_ref, flx_ref, fly_ref, flz_ref,
          v0_ref, g0_ref, b0_ref, v1_ref, g1_ref, b1_ref,
          v2_ref, g2_ref, b2_ref, xyz_ref, o_ref,
          M_s, W1_s, W2_s):
    f32 = jnp.float32

    @pl.when(pl.program_id(0) == 0)
    def _init():
        v0 = v0_ref[...]
        W0 = v0 * (g0_ref[...] * jax.lax.rsqrt(
            jnp.sum(v0 * v0, axis=1, keepdims=True)))
        E = E_ref[...]
        for a, fl_ref in enumerate((flx_ref, fly_ref, flz_ref)):
            tab = jnp.dot(fl_ref[...], E, preferred_element_type=f32)
            W0a = jnp.concatenate(
                [W0[:, _C * a:_C * a + _C],
                 W0[:, 3 * _C + _C * a:3 * _C + _C * a + _C]], axis=1)
            MaT = jax.lax.dot_general(
                W0a, tab, (((1,), (1,)), ((), ())),
                preferred_element_type=f32)
            M_s[:, _L * a:_L * a + _L] = MaT
        v1 = v1_ref[...]
        W1_s[...] = v1 * (g1_ref[...] * jax.lax.rsqrt(
            jnp.sum(v1 * v1, axis=1, keepdims=True)))
        v2 = v2_ref[...]
        W2_s[...] = v2 * (g2_ref[...] * jax.lax.rsqrt(
            jnp.sum(v2 * v2, axis=1, keepdims=True)))

    p = jnp.clip(xyz_ref[...], 0.0, 1.0) * (_L - 1.0)
    iot = jax.lax.broadcasted_iota(jnp.int32, (_L, _B), 0).astype(f32)
    hats = [jnp.maximum(1.0 - jnp.abs(p[a:a + 1, :] - iot), 0.0)
            for a in range(3)]
    Hall = jnp.concatenate(hats, axis=0)
    h = jnp.dot(M_s[...], Hall, preferred_element_type=f32) + b0_ref[...]
    h = jnp.maximum(h, 0.0)
    h = jnp.dot(W1_s[...], h, preferred_element_type=f32) + b1_ref[...]
    h = jnp.maximum(h, 0.0)
    o_ref[...] = jnp.dot(W2_s[...], h, preferred_element_type=f32) + b2_ref[...]


@jax.jit
def kernel(expr, jaw_quat_weight, xyz, feat_lines_x, feat_lines_y,
           feat_lines_z, v0, g0, b0, v1, g1, b1, v2, g2, b2):
    f32 = jnp.float32
    n = xyz.shape[0]
    e = expr.reshape(-1)[:_EXPR]
    jw = jaw_quat_weight.reshape(-1)
    eye = jnp.eye(_C, dtype=f32)
    Ebs = (e[:, None, None] * eye).reshape(_EXPR * _C, _C)
    Ejw = (jw[:, None, None] * eye).reshape(jw.shape[0] * _C, _C)
    E = jnp.concatenate([
        jnp.concatenate([Ebs, jnp.zeros_like(Ebs)], axis=1),
        jnp.concatenate([jnp.zeros_like(Ejw), Ejw], axis=1)], axis=0)

    def lines2d(fl):
        return fl.transpose(1, 0, 2).reshape(_L, fl.shape[0] * _C)

    grid = n // _B
    whole = lambda shp: pl.BlockSpec(shp, lambda i: (0,) * len(shp))
    out = pl.pallas_call(
        _body,
        grid=(grid,),
        in_specs=[
            whole(E.shape),
            whole((_L, 96 * _C)), whole((_L, 96 * _C)), whole((_L, 96 * _C)),
            whole(v0.shape), whole((v0.shape[0], 1)), whole((v0.shape[0], 1)),
            whole(v1.shape), whole((v1.shape[0], 1)), whole((v1.shape[0], 1)),
            whole(v2.shape), whole((1, 1)), whole((1, 1)),
            pl.BlockSpec((3, _B), lambda i: (0, i)),
        ],
        out_specs=pl.BlockSpec((1, _B), lambda i: (0, i)),
        out_shape=jax.ShapeDtypeStruct((1, n), f32),
        scratch_shapes=[
            pltpu.VMEM((128, 3 * _L), f32),
            pltpu.VMEM((128, 128), f32),
            pltpu.VMEM((1, 128), f32),
        ],
        compiler_params=pltpu.CompilerParams(
            dimension_semantics=("arbitrary",)),
    )(E, lines2d(feat_lines_x), lines2d(feat_lines_y), lines2d(feat_lines_z),
      v0, g0.reshape(-1, 1), b0.reshape(-1, 1),
      v1, g1.reshape(-1, 1), b1.reshape(-1, 1),
      v2, g2.reshape(1, 1), b2.reshape(1, 1),
      xyz.T)
    return out.reshape(n, 1)
